# Initial kernel scaffold; baseline (speedup 1.0000x reference)
import jax
import jax.numpy as jnp
from jax.experimental import pallas as pl


def kernel(features, W1, al1, ar1, resW1, b1, W2, al2, ar2, resW2, b2, W3, al3, ar3, resW3, b3, edge_index):
    raise NotImplementedError("write your pallas kernel here")


# speedup vs baseline: 33.7266x; 33.7266x over previous
import functools

import jax
import jax.numpy as jnp
from jax import lax
from jax.experimental import pallas as pl
from jax.experimental.pallas import tpu as pltpu
from jax.experimental.pallas import tpu_sc as plsc

N = 10000
E = 320000
D = 128
H = 4
NS = 16
NW = 32

BN = 400
NBLK = N // BN

SC1_CH = 512
SC1_NCH = E // SC1_CH
SC1_BASE_N = SC1_NCH // NW
SC1_REM = SC1_NCH - SC1_BASE_N * NW

SC2_CH = 128
SC2_NCH = E // SC2_CH
SC2_BASE_N = SC2_NCH // NS
SC2_REM = SC2_NCH - SC2_BASE_N * NS

SC2_ROWS = 624


def _tc_a_body(x_ref, w_ref, b_ref, rwm_ref, feat_ref, elr_ref, resm_ref):
    x = x_ref[...]
    y = lax.dot_general(x, w_ref[...], (((1,), (1,)), ((), ())),
                        preferred_element_type=jnp.float32)
    for h in range(H):
        feat_ref[h] = y[:, h * D:(h + 1) * D]
    elr_ref[...] = jnp.dot(x, b_ref[...], preferred_element_type=jnp.float32)
    resm_ref[...] = lax.dot_general(x, rwm_ref[...], (((1,), (1,)), ((), ())),
                                    preferred_element_type=jnp.float32)


def _tc_a(x, w, b, rwm):
    return pl.pallas_call(
        _tc_a_body,
        grid=(NBLK,),
        in_specs=[
            pl.BlockSpec((BN, D), lambda i: (i, 0)),
            pl.BlockSpec((H * D, D), lambda i: (0, 0)),
            pl.BlockSpec((D, 2 * H), lambda i: (0, 0)),
            pl.BlockSpec((D, D), lambda i: (0, 0)),
        ],
        out_specs=[
            pl.BlockSpec((H, BN, D), lambda i: (0, i, 0)),
            pl.BlockSpec((BN, 2 * H), lambda i: (i, 0)),
            pl.BlockSpec((BN, D), lambda i: (i, 0)),
        ],
        out_shape=[
            jax.ShapeDtypeStruct((H, N, D), jnp.float32),
            jax.ShapeDtypeStruct((N, 2 * H), jnp.float32),
            jax.ShapeDtypeStruct((N, D), jnp.float32),
        ],
    )(x, w, b, rwm)


def _sc1_body(src_hbm, dst_hbm, elr_hbm, denp_hbm, w_hbm,
              tab, acc, src_c, dst_c, w_c):
    c = lax.axis_index("c")
    s = lax.axis_index("s")
    wid = c * NS + s

    def zbody(i, _):
        acc[pl.ds(i * 16, 16)] = jnp.zeros((16,), jnp.float32)
        return 0
    lax.fori_loop(0, N * H // 16, zbody, 0)

    pltpu.sync_copy(elr_hbm, tab)

    nch = SC1_BASE_N + jnp.where(wid < SC1_REM, 1, 0)
    base = wid * SC1_BASE_N + jnp.minimum(wid, SC1_REM)

    def chunk_body(j, _):
        be = (base + j) * SC1_CH
        pltpu.sync_copy(src_hbm.at[pl.ds(be, SC1_CH)], src_c)
        pltpu.sync_copy(dst_hbm.at[pl.ds(be, SC1_CH)], dst_c)

        def vec_body(i, _):
            s16 = src_c[pl.ds(i * 16, 16)] * (2 * H)
            d16 = dst_c[pl.ds(i * 16, 16)]
            d16h = d16 * (2 * H)
            d16a = d16 * H
            for h in range(H):
                el = plsc.load_gather(tab, [s16 + h])
                er = plsc.load_gather(tab, [d16h + (H + h)])
                x = el + er
                wv = jnp.exp(jnp.maximum(x, 0.2 * x))
                w_c[h, pl.ds(i * 16, 16)] = wv
                plsc.addupdate_scatter(acc, [d16a + h], wv)
            return 0
        lax.fori_loop(0, SC1_CH // 16, vec_body, 0)
        for h in range(H):
            pltpu.sync_copy(w_c.at[h], w_hbm.at[pl.ds(h * E + be, SC1_CH)])
        return 0
    lax.fori_loop(0, nch, chunk_body, 0)

    pltpu.sync_copy(acc, denp_hbm.at[pl.ds(wid * N * H, N * H)])


def _sc1(src, dst, elr):
    mesh = plsc.VectorSubcoreMesh(core_axis_name="c", subcore_axis_name="s")
    f = pl.kernel(
        _sc1_body,
        out_type=[
            jax.ShapeDtypeStruct((NW * N * H,), jnp.float32),
            jax.ShapeDtypeStruct((H * E,), jnp.float32),
        ],
        mesh=mesh,
        scratch_types=[
            pltpu.VMEM((N * 2 * H,), jnp.float32),
            pltpu.VMEM((N * H,), jnp.float32),
            pltpu.VMEM((SC1_CH,), jnp.int32),
            pltpu.VMEM((SC1_CH,), jnp.int32),
            pltpu.VMEM((H, SC1_CH), jnp.float32),
        ],
        compiler_params=pltpu.CompilerParams(needs_layout_passes=False),
    )
    return f(src, dst, elr)


def _sc2_body(src_hbm, dst_hbm, w_hbm, feat_hbm, rstw_hbm,
              src_c, dst_c0, dst_c1, w_c0, w_c1, idx0, idx1,
              fbuf0, fbuf1, zbuf, acc_sh, sem0, sem1, ssem0, ssem1):
    c = lax.axis_index("c")
    s = lax.axis_index("s")

    def zb(i, _):
        for j in range(8):
            zbuf[i, pl.ds(j * 16, 16)] = jnp.zeros((16,), jnp.float32)
        return 0
    lax.fori_loop(0, 104, zb, 0)

    nch = SC2_BASE_N + jnp.where(s < SC2_REM, 1, 0)
    base = s * SC2_BASE_N + jnp.minimum(s, SC2_REM)
    roff = s * SC2_ROWS
    last = s == NS - 1

    for k in range(2):
        h = c * 2 + k
        hoff = (h * N).astype(jnp.int32)

        for slab in range(6):
            pltpu.sync_copy(zbuf, acc_sh.at[pl.ds(roff + slab * 104, 104), :])

        @pl.when(last)
        def _():
            pltpu.sync_copy(zbuf.at[pl.ds(0, 16), :],
                            acc_sh.at[pl.ds(N - 16, 16), :])
        plsc.subcore_barrier()

        def drain_scatter(dst_c, fbuf, ssem):
            pltpu.make_async_copy(fbuf, acc_sh.at[dst_c], ssem).wait()

        def issue(j, dst_c, w_c, idx_c, fbuf, sem, ssem):
            @pl.when(j >= 2)
            def _():
                drain_scatter(dst_c, fbuf, ssem)
            be = (base + j) * SC2_CH
            pltpu.sync_copy(src_hbm.at[pl.ds(be, SC2_CH)], src_c)
            pltpu.sync_copy(dst_hbm.at[pl.ds(be, SC2_CH)], dst_c)
            pltpu.sync_copy(w_hbm.at[pl.ds(h * E + be, SC2_CH)], w_c)
            def ib(i, _):
                idx_c[pl.ds(i * 16, 16)] = src_c[pl.ds(i * 16, 16)] + hoff
                return 0
            lax.fori_loop(0, SC2_CH // 16, ib, 0)
            pltpu.async_copy(feat_hbm.at[idx_c], fbuf, sem)

        def process(dst_c, w_c, idx_c, fbuf, sem, ssem):
            pltpu.make_async_copy(feat_hbm.at[idx_c], fbuf, sem).wait()
            def scale(i, _):
                for u in range(8):
                    kk = i * 8 + u
                    av = plsc.load_gather(w_c, [jnp.full((16,), kk, jnp.int32)])
                    for j in range(8):
                        fbuf[kk, pl.ds(j * 16, 16)] = (
                            fbuf[kk, pl.ds(j * 16, 16)] * av)
                return 0
            lax.fori_loop(0, SC2_CH // 8, scale, 0)
            pltpu.async_copy(fbuf, acc_sh.at[dst_c], ssem, add=True)

        issue(0, dst_c0, w_c0, idx0, fbuf0, sem0, ssem0)

        def pair_body(p, _):
            j0 = 2 * p

            @pl.when(j0 + 1 < nch)
            def _():
                issue(j0 + 1, dst_c1, w_c1, idx1, fbuf1, sem1, ssem1)
            process(dst_c0, w_c0, idx0, fbuf0, sem0, ssem0)

            @pl.when(j0 + 2 < nch)
            def _():
                issue(j0 + 2, dst_c0, w_c0, idx0, fbuf0, sem0, ssem0)

            @pl.when(j0 + 1 < nch)
            def _():
                process(dst_c1, w_c1, idx1, fbuf1, sem1, ssem1)
            return 0
        lax.fori_loop(0, (nch + 1) // 2, pair_body, 0)

        drain_scatter(dst_c0, fbuf0, ssem0)
        @pl.when(nch >= 2)
        def _():
            drain_scatter(dst_c1, fbuf1, ssem1)
        plsc.subcore_barrier()

        pltpu.sync_copy(acc_sh.at[pl.ds(roff, SC2_ROWS), :],
                        rstw_hbm.at[pl.ds(h * N + roff, SC2_ROWS), :])

        @pl.when(last)
        def _():
            pltpu.sync_copy(acc_sh.at[pl.ds(N - 16, 16), :],
                            rstw_hbm.at[pl.ds(h * N + N - 16, 16), :])
        plsc.subcore_barrier()


def _sc2(src, dst, w4, feat_flat):
    mesh = plsc.VectorSubcoreMesh(core_axis_name="c", subcore_axis_name="s")
    f = pl.kernel(
        _sc2_body,
        out_type=jax.ShapeDtypeStruct((H * N, D), jnp.float32),
        mesh=mesh,
        scratch_types=[
            pltpu.VMEM((SC2_CH,), jnp.int32),
            pltpu.VMEM((SC2_CH,), jnp.int32),
            pltpu.VMEM((SC2_CH,), jnp.int32),
            pltpu.VMEM((SC2_CH,), jnp.float32),
            pltpu.VMEM((SC2_CH,), jnp.float32),
            pltpu.VMEM((SC2_CH,), jnp.int32),
            pltpu.VMEM((SC2_CH,), jnp.int32),
            pltpu.VMEM((SC2_CH, D), jnp.float32),
            pltpu.VMEM((SC2_CH, D), jnp.float32),
            pltpu.VMEM((104, D), jnp.float32),
            pltpu.VMEM_SHARED((N, D), jnp.float32),
            pltpu.SemaphoreType.DMA,
            pltpu.SemaphoreType.DMA,
            pltpu.SemaphoreType.DMA,
            pltpu.SemaphoreType.DMA,
        ],
        compiler_params=pltpu.CompilerParams(needs_layout_passes=False),
    )
    return f(src, dst, w4, feat_flat)


def _tc_b_body(rstw_ref, denp_ref, resm_ref, bm_ref, out_ref, *, relu):
    dsum = jnp.sum(denp_ref[...], axis=0)
    inv = 1.0 / jnp.where(dsum > 0.0, dsum, 1.0)
    acc = jnp.zeros_like(resm_ref[...])
    for h in range(H):
        acc = acc + rstw_ref[h] * inv[:, h][:, None]
    acc = acc * (1.0 / H) + resm_ref[...] + bm_ref[...]
    if relu:
        acc = jnp.maximum(acc, 0.0)
    out_ref[...] = acc


def _tc_b_final_body(rstw_ref, denp_ref, resm_ref, bm_ref, out_ref, ge_ref):
    dsum = jnp.sum(denp_ref[...], axis=0)
    inv = 1.0 / jnp.where(dsum > 0.0, dsum, 1.0)
    acc = jnp.zeros_like(resm_ref[...])
    for h in range(H):
        acc = acc + rstw_ref[h] * inv[:, h][:, None]
    acc = acc * (1.0 / H) + resm_ref[...] + bm_ref[...]
    out_ref[...] = acc

    @pl.when(pl.program_id(0) == 0)
    def _():
        ge_ref[...] = jnp.zeros_like(ge_ref)
    ge_ref[...] += jnp.sum(acc, axis=0, keepdims=True) * (1.0 / N)


_TCB_IN_SPECS = [
    pl.BlockSpec((H, BN, D), lambda i: (0, i, 0)),
    pl.BlockSpec((NW, BN, H), lambda i: (0, i, 0)),
    pl.BlockSpec((BN, D), lambda i: (i, 0)),
    pl.BlockSpec((1, D), lambda i: (0, 0)),
]


def _tc_b(rstw, denp, resm, bm, relu):
    return pl.pallas_call(
        functools.partial(_tc_b_body, relu=relu),
        grid=(NBLK,),
        in_specs=_TCB_IN_SPECS,
        out_specs=pl.BlockSpec((BN, D), lambda i: (i, 0)),
        out_shape=jax.ShapeDtypeStruct((N, D), jnp.float32),
    )(rstw, denp, resm, bm)


def _tc_b_final(rstw, denp, resm, bm):
    return pl.pallas_call(
        _tc_b_final_body,
        grid=(NBLK,),
        in_specs=_TCB_IN_SPECS,
        out_specs=[
            pl.BlockSpec((BN, D), lambda i: (i, 0)),
            pl.BlockSpec((1, D), lambda i: (0, 0)),
        ],
        out_shape=[
            jax.ShapeDtypeStruct((N, D), jnp.float32),
            jax.ShapeDtypeStruct((1, D), jnp.float32),
        ],
    )(rstw, denp, resm, bm)


def _layer(h, src, dst, W, al, ar, resW, b, relu, final):
    Wr = W.reshape(H, D, D)
    B = jnp.concatenate(
        [jnp.einsum("hdk,hd->kh", Wr, al), jnp.einsum("hdk,hd->kh", Wr, ar)],
        axis=1)
    resWm = resW.reshape(H, D, D).mean(axis=0)
    bm = b.reshape(H, D).mean(axis=0).reshape(1, D)

    feat, elr, resm = _tc_a(h, W, B, resWm)
    denp, w4 = _sc1(src, dst, elr.reshape(N * 2 * H))
    rstw = _sc2(src, dst, w4, feat.reshape(H * N, D)).reshape(H, N, D)
    denp = denp.reshape(NW, N, H)
    if final:
        return _tc_b_final(rstw, denp, resm, bm)
    return _tc_b(rstw, denp, resm, bm, relu), None


def kernel(features, W1, al1, ar1, resW1, b1, W2, al2, ar2, resW2, b2,
           W3, al3, ar3, resW3, b3, edge_index):
    src = edge_index[0]
    dst = edge_index[1]
    h, _ = _layer(features, src, dst, W1, al1, ar1, resW1, b1, True, False)
    h, _ = _layer(h, src, dst, W2, al2, ar2, resW2, b2, True, False)
    h, ge = _layer(h, src, dst, W3, al3, ar3, resW3, b3, False, True)
    return (h, ge.reshape(D))


# speedup vs baseline: 36.5123x; 1.0826x over previous
import functools

import jax
import jax.numpy as jnp
from jax import lax
from jax.experimental import pallas as pl
from jax.experimental.pallas import tpu as pltpu
from jax.experimental.pallas import tpu_sc as plsc

N = 10000
E = 320000
D = 128
H = 4
NS = 16
NW = 32

BN = 400
NBLK = N // BN

SC1_CH = 512
SC1_NCH = E // SC1_CH
SC1_BASE_N = SC1_NCH // NW
SC1_REM = SC1_NCH - SC1_BASE_N * NW

SC2_CH = 160
SC2_NCH = E // SC2_CH
SC2_BASE_N = SC2_NCH // NS
SC2_REM = SC2_NCH - SC2_BASE_N * NS

SC2_ROWS = 624


def _tc_a_body(x_ref, w_ref, b_ref, rwm_ref, feat_ref, elr_ref, resm_ref):
    x = x_ref[...]
    y = lax.dot_general(x, w_ref[...], (((1,), (1,)), ((), ())),
                        preferred_element_type=jnp.float32)
    for h in range(H):
        feat_ref[h] = y[:, h * D:(h + 1) * D]
    elr_ref[...] = jnp.dot(x, b_ref[...], preferred_element_type=jnp.float32)
    resm_ref[...] = lax.dot_general(x, rwm_ref[...], (((1,), (1,)), ((), ())),
                                    preferred_element_type=jnp.float32)


def _tc_a(x, w, b, rwm):
    return pl.pallas_call(
        _tc_a_body,
        grid=(NBLK,),
        in_specs=[
            pl.BlockSpec((BN, D), lambda i: (i, 0)),
            pl.BlockSpec((H * D, D), lambda i: (0, 0)),
            pl.BlockSpec((D, 2 * H), lambda i: (0, 0)),
            pl.BlockSpec((D, D), lambda i: (0, 0)),
        ],
        out_specs=[
            pl.BlockSpec((H, BN, D), lambda i: (0, i, 0)),
            pl.BlockSpec((BN, 2 * H), lambda i: (i, 0)),
            pl.BlockSpec((BN, D), lambda i: (i, 0)),
        ],
        out_shape=[
            jax.ShapeDtypeStruct((H, N, D), jnp.float32),
            jax.ShapeDtypeStruct((N, 2 * H), jnp.float32),
            jax.ShapeDtypeStruct((N, D), jnp.float32),
        ],
    )(x, w, b, rwm)


def _sc1_body(src_hbm, dst_hbm, elr_hbm, denp_hbm, w_hbm,
              tab, acc, src_c, dst_c, w_c):
    c = lax.axis_index("c")
    s = lax.axis_index("s")
    wid = c * NS + s

    def zbody(i, _):
        acc[pl.ds(i * 16, 16)] = jnp.zeros((16,), jnp.float32)
        return 0
    lax.fori_loop(0, N * H // 16, zbody, 0)

    pltpu.sync_copy(elr_hbm, tab)

    nch = SC1_BASE_N + jnp.where(wid < SC1_REM, 1, 0)
    base = wid * SC1_BASE_N + jnp.minimum(wid, SC1_REM)

    def chunk_body(j, _):
        be = (base + j) * SC1_CH
        pltpu.sync_copy(src_hbm.at[pl.ds(be, SC1_CH)], src_c)
        pltpu.sync_copy(dst_hbm.at[pl.ds(be, SC1_CH)], dst_c)

        def vec_body(i, _):
            s16 = src_c[pl.ds(i * 16, 16)] * (2 * H)
            d16 = dst_c[pl.ds(i * 16, 16)]
            d16h = d16 * (2 * H)
            d16a = d16 * H
            for h in range(H):
                el = plsc.load_gather(tab, [s16 + h])
                er = plsc.load_gather(tab, [d16h + (H + h)])
                x = el + er
                wv = jnp.exp(jnp.maximum(x, 0.2 * x))
                w_c[h, pl.ds(i * 16, 16)] = wv
                plsc.addupdate_scatter(acc, [d16a + h], wv)
            return 0
        lax.fori_loop(0, SC1_CH // 16, vec_body, 0)
        for h in range(H):
            pltpu.sync_copy(w_c.at[h], w_hbm.at[pl.ds(h * E + be, SC1_CH)])
        return 0
    lax.fori_loop(0, nch, chunk_body, 0)

    pltpu.sync_copy(acc, denp_hbm.at[pl.ds(wid * N * H, N * H)])


def _sc1(src, dst, elr):
    mesh = plsc.VectorSubcoreMesh(core_axis_name="c", subcore_axis_name="s")
    f = pl.kernel(
        _sc1_body,
        out_type=[
            jax.ShapeDtypeStruct((NW * N * H,), jnp.float32),
            jax.ShapeDtypeStruct((H * E,), jnp.float32),
        ],
        mesh=mesh,
        scratch_types=[
            pltpu.VMEM((N * 2 * H,), jnp.float32),
            pltpu.VMEM((N * H,), jnp.float32),
            pltpu.VMEM((SC1_CH,), jnp.int32),
            pltpu.VMEM((SC1_CH,), jnp.int32),
            pltpu.VMEM((H, SC1_CH), jnp.float32),
        ],
        compiler_params=pltpu.CompilerParams(needs_layout_passes=False),
    )
    return f(src, dst, elr)


def _sc2_body(src_hbm, dst_hbm, w_hbm, feat_hbm, rstw_hbm,
              src_c, dst_c0, dst_c1, w_c0, w_c1, idx0, idx1,
              fbuf0, fbuf1, zbuf, acc_sh, sem0, sem1, ssem0, ssem1):
    c = lax.axis_index("c")
    s = lax.axis_index("s")

    def zb(i, _):
        for j in range(8):
            zbuf[i, pl.ds(j * 16, 16)] = jnp.zeros((16,), jnp.float32)
        return 0
    lax.fori_loop(0, 24, zb, 0)

    nch = SC2_BASE_N + jnp.where(s < SC2_REM, 1, 0)
    base = s * SC2_BASE_N + jnp.minimum(s, SC2_REM)
    roff = s * SC2_ROWS
    last = s == NS - 1

    for k in range(2):
        h = c * 2 + k
        hoff = (h * N).astype(jnp.int32)

        for slab in range(26):
            pltpu.sync_copy(zbuf, acc_sh.at[pl.ds(roff + slab * 24, 24), :])

        @pl.when(last)
        def _():
            pltpu.sync_copy(zbuf.at[pl.ds(0, 16), :],
                            acc_sh.at[pl.ds(N - 16, 16), :])
        plsc.subcore_barrier()

        def drain_scatter(dst_c, fbuf, ssem):
            pltpu.make_async_copy(fbuf, acc_sh.at[dst_c], ssem).wait()

        def issue(j, dst_c, w_c, idx_c, fbuf, sem, ssem):
            @pl.when(j >= 2)
            def _():
                drain_scatter(dst_c, fbuf, ssem)
            be = (base + j) * SC2_CH
            pltpu.sync_copy(src_hbm.at[pl.ds(be, SC2_CH)], src_c)
            pltpu.sync_copy(dst_hbm.at[pl.ds(be, SC2_CH)], dst_c)
            pltpu.sync_copy(w_hbm.at[pl.ds(h * E + be, SC2_CH)], w_c)
            def ib(i, _):
                idx_c[pl.ds(i * 16, 16)] = src_c[pl.ds(i * 16, 16)] + hoff
                return 0
            lax.fori_loop(0, SC2_CH // 16, ib, 0)
            pltpu.async_copy(feat_hbm.at[idx_c], fbuf, sem)

        def process(dst_c, w_c, idx_c, fbuf, sem, ssem):
            pltpu.make_async_copy(feat_hbm.at[idx_c], fbuf, sem).wait()
            def scale(i, _):
                for u in range(8):
                    kk = i * 8 + u
                    av = plsc.load_gather(w_c, [jnp.full((16,), kk, jnp.int32)])
                    for j in range(8):
                        fbuf[kk, pl.ds(j * 16, 16)] = (
                            fbuf[kk, pl.ds(j * 16, 16)] * av)
                return 0
            lax.fori_loop(0, SC2_CH // 8, scale, 0)
            pltpu.async_copy(fbuf, acc_sh.at[dst_c], ssem, add=True)

        issue(0, dst_c0, w_c0, idx0, fbuf0, sem0, ssem0)

        def pair_body(p, _):
            j0 = 2 * p

            @pl.when(j0 + 1 < nch)
            def _():
                issue(j0 + 1, dst_c1, w_c1, idx1, fbuf1, sem1, ssem1)
            process(dst_c0, w_c0, idx0, fbuf0, sem0, ssem0)

            @pl.when(j0 + 2 < nch)
            def _():
                issue(j0 + 2, dst_c0, w_c0, idx0, fbuf0, sem0, ssem0)

            @pl.when(j0 + 1 < nch)
            def _():
                process(dst_c1, w_c1, idx1, fbuf1, sem1, ssem1)
            return 0
        lax.fori_loop(0, (nch + 1) // 2, pair_body, 0)

        drain_scatter(dst_c0, fbuf0, ssem0)
        @pl.when(nch >= 2)
        def _():
            drain_scatter(dst_c1, fbuf1, ssem1)
        plsc.subcore_barrier()

        pltpu.sync_copy(acc_sh.at[pl.ds(roff, SC2_ROWS), :],
                        rstw_hbm.at[pl.ds(h * N + roff, SC2_ROWS), :])

        @pl.when(last)
        def _():
            pltpu.sync_copy(acc_sh.at[pl.ds(N - 16, 16), :],
                            rstw_hbm.at[pl.ds(h * N + N - 16, 16), :])
        plsc.subcore_barrier()


def _sc2(src, dst, w4, feat_flat):
    mesh = plsc.VectorSubcoreMesh(core_axis_name="c", subcore_axis_name="s")
    f = pl.kernel(
        _sc2_body,
        out_type=jax.ShapeDtypeStruct((H * N, D), jnp.float32),
        mesh=mesh,
        scratch_types=[
            pltpu.VMEM((SC2_CH,), jnp.int32),
            pltpu.VMEM((SC2_CH,), jnp.int32),
            pltpu.VMEM((SC2_CH,), jnp.int32),
            pltpu.VMEM((SC2_CH,), jnp.float32),
            pltpu.VMEM((SC2_CH,), jnp.float32),
            pltpu.VMEM((SC2_CH,), jnp.int32),
            pltpu.VMEM((SC2_CH,), jnp.int32),
            pltpu.VMEM((SC2_CH, D), jnp.float32),
            pltpu.VMEM((SC2_CH, D), jnp.float32),
            pltpu.VMEM((24, D), jnp.float32),
            pltpu.VMEM_SHARED((N, D), jnp.float32),
            pltpu.SemaphoreType.DMA,
            pltpu.SemaphoreType.DMA,
            pltpu.SemaphoreType.DMA,
            pltpu.SemaphoreType.DMA,
        ],
        compiler_params=pltpu.CompilerParams(needs_layout_passes=False),
    )
    return f(src, dst, w4, feat_flat)


def _tc_b_body(rstw_ref, denp_ref, resm_ref, bm_ref, out_ref, *, relu):
    dsum = jnp.sum(denp_ref[...], axis=0)
    inv = 1.0 / jnp.where(dsum > 0.0, dsum, 1.0)
    acc = jnp.zeros_like(resm_ref[...])
    for h in range(H):
        acc = acc + rstw_ref[h] * inv[:, h][:, None]
    acc = acc * (1.0 / H) + resm_ref[...] + bm_ref[...]
    if relu:
        acc = jnp.maximum(acc, 0.0)
    out_ref[...] = acc


def _tc_b_final_body(rstw_ref, denp_ref, resm_ref, bm_ref, out_ref, ge_ref):
    dsum = jnp.sum(denp_ref[...], axis=0)
    inv = 1.0 / jnp.where(dsum > 0.0, dsum, 1.0)
    acc = jnp.zeros_like(resm_ref[...])
    for h in range(H):
        acc = acc + rstw_ref[h] * inv[:, h][:, None]
    acc = acc * (1.0 / H) + resm_ref[...] + bm_ref[...]
    out_ref[...] = acc

    @pl.when(pl.program_id(0) == 0)
    def _():
        ge_ref[...] = jnp.zeros_like(ge_ref)
    ge_ref[...] += jnp.sum(acc, axis=0, keepdims=True) * (1.0 / N)


_TCB_IN_SPECS = [
    pl.BlockSpec((H, BN, D), lambda i: (0, i, 0)),
    pl.BlockSpec((NW, BN, H), lambda i: (0, i, 0)),
    pl.BlockSpec((BN, D), lambda i: (i, 0)),
    pl.BlockSpec((1, D), lambda i: (0, 0)),
]


def _tc_b(rstw, denp, resm, bm, relu):
    return pl.pallas_call(
        functools.partial(_tc_b_body, relu=relu),
        grid=(NBLK,),
        in_specs=_TCB_IN_SPECS,
        out_specs=pl.BlockSpec((BN, D), lambda i: (i, 0)),
        out_shape=jax.ShapeDtypeStruct((N, D), jnp.float32),
    )(rstw, denp, resm, bm)


def _tc_b_final(rstw, denp, resm, bm):
    return pl.pallas_call(
        _tc_b_final_body,
        grid=(NBLK,),
        in_specs=_TCB_IN_SPECS,
        out_specs=[
            pl.BlockSpec((BN, D), lambda i: (i, 0)),
            pl.BlockSpec((1, D), lambda i: (0, 0)),
        ],
        out_shape=[
            jax.ShapeDtypeStruct((N, D), jnp.float32),
            jax.ShapeDtypeStruct((1, D), jnp.float32),
        ],
    )(rstw, denp, resm, bm)


def _layer(h, src, dst, W, al, ar, resW, b, relu, final):
    Wr = W.reshape(H, D, D)
    B = jnp.concatenate(
        [jnp.einsum("hdk,hd->kh", Wr, al), jnp.einsum("hdk,hd->kh", Wr, ar)],
        axis=1)
    resWm = resW.reshape(H, D, D).mean(axis=0)
    bm = b.reshape(H, D).mean(axis=0).reshape(1, D)

    feat, elr, resm = _tc_a(h, W, B, resWm)
    denp, w4 = _sc1(src, dst, elr.reshape(N * 2 * H))
    rstw = _sc2(src, dst, w4, feat.reshape(H * N, D)).reshape(H, N, D)
    denp = denp.reshape(NW, N, H)
    if final:
        return _tc_b_final(rstw, denp, resm, bm)
    return _tc_b(rstw, denp, resm, bm, relu), None


def kernel(features, W1, al1, ar1, resW1, b1, W2, al2, ar2, resW2, b2,
           W3, al3, ar3, resW3, b3, edge_index):
    src = edge_index[0]
    dst = edge_index[1]
    h, _ = _layer(features, src, dst, W1, al1, ar1, resW1, b1, True, False)
    h, _ = _layer(h, src, dst, W2, al2, ar2, resW2, b2, True, False)
    h, ge = _layer(h, src, dst, W3, al3, ar3, resW3, b3, False, True)
    return (h, ge.reshape(D))
